# Initial kernel scaffold; baseline (speedup 1.0000x reference)
import jax
import jax.numpy as jnp
from jax.experimental import pallas as pl


def kernel(attention_weights, token_to_sentence_map):
    raise NotImplementedError("write your pallas kernel here")


# speedup vs baseline: 12.4883x; 12.4883x over previous
import jax
import jax.numpy as jnp
from jax import lax
from jax.experimental import pallas as pl

_B, _NH, _T = 4, 16, 8192
_S = 256
_K = 5
_CHUNK = 1024


def _body(attn_ref, map_ref, vals_ref, idx_ref):
    acc = jnp.zeros((_B, _S), jnp.float32)
    for c in range(_T // _CHUNK):
        sl = pl.ds(c * _CHUNK, _CHUNK)
        avg = jnp.sum(attn_ref[:, :, sl], axis=1) * (1.0 / _NH)
        m = map_ref[:, sl]
        iota = lax.broadcasted_iota(jnp.int32, (_S, _CHUNK), 0)
        onehot = jnp.where(m == iota, 1.0, 0.0).astype(jnp.float32)
        acc = acc + lax.dot_general(
            avg, onehot,
            dimension_numbers=(((1,), (1,)), ((), ())),
            preferred_element_type=jnp.float32,
        )
    total = jnp.sum(acc, axis=-1, keepdims=True)
    scores = acc / total

    col = lax.broadcasted_iota(jnp.int32, (_B, 8), 1)
    sent = lax.broadcasted_iota(jnp.int32, (_B, _S), 1)
    vals_acc = jnp.zeros((_B, 8), jnp.float32)
    idx_acc = jnp.zeros((_B, 8), jnp.int32)
    work = scores
    for i in range(_K):
        mx = jnp.max(work, axis=-1, keepdims=True)
        cand = jnp.where(work == mx, sent, jnp.int32(1 << 30))
        ind = jnp.min(cand, axis=-1, keepdims=True)
        vals_acc = jnp.where(col == i, mx, vals_acc)
        idx_acc = jnp.where(col == i, ind, idx_acc)
        work = jnp.where(sent == ind, jnp.float32(-1.0), work)
    vals_ref[...] = vals_acc
    idx_ref[...] = idx_acc


def kernel(attention_weights, token_to_sentence_map):
    map2d = token_to_sentence_map.astype(jnp.int32).reshape(1, _T)
    vals, idx = pl.pallas_call(
        _body,
        out_shape=[
            jax.ShapeDtypeStruct((_B, 8), jnp.float32),
            jax.ShapeDtypeStruct((_B, 8), jnp.int32),
        ],
    )(attention_weights, map2d)
    return vals[:, :_K], idx[:, :_K]
